# Initial kernel scaffold; baseline (speedup 1.0000x reference)
import functools

import jax
import jax.numpy as jnp
from jax import lax
from jax.experimental import pallas as pl
from jax.experimental.pallas import tpu as pltpu
from jax.experimental.pallas import tpu_sc as plsc

NUM_USERS = 20000
NUM_ITEMS = 30000
N_NODES = NUM_USERS + NUM_ITEMS
D = 64
N_LAYERS = 3

HALF = N_NODES // 2
HPAD = 25088
NPAD = 2 * HPAD
SHIFT = HPAD - HALF
GARBAGE = HALF + 8

CHUNK = 128
ROWS_PER_TILE = HPAD // 16


def _layer_body(table, src, dst, wgt, zrows, out, idxs, idxd, wv, rows, sem):
    cid = lax.axis_index("c")
    sid = lax.axis_index("s")

    def inner(acc):
        base = cid * HALF
        pltpu.sync_copy(zrows, acc.at[pl.ds(sid * ROWS_PER_TILE, ROWS_PER_TILE)])
        plsc.subcore_barrier()

        ep_tile = src.shape[0] // 16
        n_chunks = ep_tile // CHUNK

        def chunk_body(ci, carry):
            off = sid * ep_tile + ci * CHUNK
            pltpu.sync_copy(src.at[pl.ds(off, CHUNK)], idxs)
            pltpu.sync_copy(dst.at[pl.ds(off, CHUNK)], idxd)
            pltpu.sync_copy(wgt.at[pl.ds(off, CHUNK)], wv)
            for j in range(CHUNK // 16):
                s = idxs[pl.ds(j * 16, 16)]
                idxs[pl.ds(j * 16, 16)] = jnp.where(s >= HALF, s + SHIFT, s)
            gcopy = pltpu.async_copy(table.at[idxs], rows, sem)
            for j in range(CHUNK // 16):
                d = idxd[pl.ds(j * 16, 16)] - base
                ok = (d >= 0) & (d < HALF)
                idxd[pl.ds(j * 16, 16)] = jnp.where(ok, d, GARBAGE)
            gcopy.wait()

            def scale_body(e, c2):
                we = wv[e]
                for k in range(D // 16):
                    rows[e, pl.ds(k * 16, 16)] = rows[e, pl.ds(k * 16, 16)] * we
                return c2

            lax.fori_loop(0, CHUNK, scale_body, 0)
            pltpu.sync_copy(rows, acc.at[idxd], add=True)
            return carry

        lax.fori_loop(0, n_chunks, chunk_body, 0)
        plsc.subcore_barrier()
        r0 = sid * ROWS_PER_TILE
        pltpu.sync_copy(acc.at[pl.ds(r0, ROWS_PER_TILE)],
                        out.at[pl.ds(cid * HPAD + r0, ROWS_PER_TILE)])

    pl.run_scoped(inner, pltpu.VMEM_SHARED((HPAD, D), jnp.float32))


def _propagate_layer(table, src, dst, wgt, zrows):
    mesh = plsc.VectorSubcoreMesh(core_axis_name="c", subcore_axis_name="s")
    return pl.kernel(
        _layer_body,
        out_type=jax.ShapeDtypeStruct((NPAD, D), jnp.float32),
        mesh=mesh,
        scratch_types=[
            pltpu.VMEM((CHUNK,), jnp.int32),
            pltpu.VMEM((CHUNK,), jnp.int32),
            pltpu.VMEM((CHUNK,), jnp.float32),
            pltpu.VMEM((CHUNK, D), jnp.float32),
            pltpu.SemaphoreType.DMA,
        ],
    )(table, src, dst, wgt, zrows)


def _mean4_body(a, b, c, d, o):
    o[...] = (a[...] + b[...] + c[...] + d[...]) * 0.25


def _mean4(t0, t1, t2, t3):
    flat = NPAD * D // 128
    blk = flat // 16
    args = [t.reshape(flat, 128) for t in (t0, t1, t2, t3)]
    spec = pl.BlockSpec((blk, 128), lambda i: (i, 0))
    out = pl.pallas_call(
        _mean4_body,
        grid=(16,),
        in_specs=[spec] * 4,
        out_specs=spec,
        out_shape=jax.ShapeDtypeStruct((flat, 128), jnp.float32),
    )(*args)
    return out.reshape(NPAD, D)


@functools.partial(jax.jit)
def kernel(user_emb, item_emb, edge_index, edge_weight):
    zpad = jnp.zeros((SHIFT, D), jnp.float32)
    table = jnp.concatenate(
        [user_emb, item_emb[: HALF - NUM_USERS], zpad,
         item_emb[HALF - NUM_USERS:], zpad], axis=0)

    n_edges = edge_index.shape[1]
    ep_tile = -(-n_edges // (16 * CHUNK)) * CHUNK
    epad = 16 * ep_tile
    pad = epad - n_edges
    src = jnp.concatenate([edge_index[0], jnp.zeros((pad,), jnp.int32)])
    dst = jnp.concatenate([edge_index[1], jnp.zeros((pad,), jnp.int32)])
    wgt = jnp.concatenate([edge_weight, jnp.zeros((pad,), jnp.float32)])
    zrows = jnp.zeros((ROWS_PER_TILE, D), jnp.float32)

    embs = [table]
    for _ in range(N_LAYERS):
        embs.append(_propagate_layer(embs[-1], src, dst, wgt, zrows))

    light_out = _mean4(*embs)
    users = light_out[:NUM_USERS]
    items = jnp.concatenate(
        [light_out[NUM_USERS:HALF], light_out[HPAD:HPAD + N_NODES - HALF]], axis=0)
    return users, items


# speedup vs baseline: 2.0051x; 2.0051x over previous
import functools

import jax
import jax.numpy as jnp
from jax import lax
from jax.experimental import pallas as pl
from jax.experimental.pallas import tpu as pltpu
from jax.experimental.pallas import tpu_sc as plsc

NUM_USERS = 20000
NUM_ITEMS = 30000
N_NODES = NUM_USERS + NUM_ITEMS
D = 64
N_LAYERS = 3

HALF = N_NODES // 2
HPAD = 25088
NPAD = 2 * HPAD
SHIFT = HPAD - HALF
GARBAGE = HALF + 8

CHUNK = 128
ROWS_PER_TILE = HPAD // 16


def _layer_body(table, src, dst, wgt, zrows, out, idxs, idxd, wv, rows, acc, sem):
    cid = lax.axis_index("c")
    sid = lax.axis_index("s")

    if True:
        base = cid * HALF
        pltpu.sync_copy(zrows, acc.at[pl.ds(sid * ROWS_PER_TILE, ROWS_PER_TILE)])
        plsc.subcore_barrier()

        ep_tile = src.shape[0] // 16
        n_chunks = ep_tile // CHUNK

        def chunk_body(ci, carry):
            off = sid * ep_tile + ci * CHUNK
            pltpu.sync_copy(src.at[pl.ds(off, CHUNK)], idxs)
            pltpu.sync_copy(dst.at[pl.ds(off, CHUNK)], idxd)
            pltpu.sync_copy(wgt.at[pl.ds(off, CHUNK)], wv)
            for j in range(CHUNK // 16):
                s = idxs[pl.ds(j * 16, 16)]
                idxs[pl.ds(j * 16, 16)] = jnp.where(s >= HALF, s + SHIFT, s)
            gcopy = pltpu.async_copy(table.at[idxs], rows, sem)
            for j in range(CHUNK // 16):
                d = idxd[pl.ds(j * 16, 16)] - base
                ok = (d >= 0) & (d < HALF)
                idxd[pl.ds(j * 16, 16)] = jnp.where(ok, d, GARBAGE)
            gcopy.wait()

            def scale_body(j, c2):
                wv16 = wv[pl.ds(j * 16, 16)]
                r0 = j * 16
                for e in range(16):
                    we = wv16[e]
                    for k in range(D // 16):
                        rows[r0 + e, pl.ds(k * 16, 16)] = (
                            rows[r0 + e, pl.ds(k * 16, 16)] * we)
                return c2

            lax.fori_loop(0, CHUNK // 16, scale_body, 0)
            pltpu.sync_copy(rows, acc.at[idxd], add=True)
            return carry

        lax.fori_loop(0, n_chunks, chunk_body, 0)
        plsc.subcore_barrier()
        r0 = sid * ROWS_PER_TILE
        pltpu.sync_copy(acc.at[pl.ds(r0, ROWS_PER_TILE)],
                        out.at[pl.ds(cid * HPAD + r0, ROWS_PER_TILE)])


def _propagate_layer(table, src, dst, wgt, zrows):
    mesh = plsc.VectorSubcoreMesh(core_axis_name="c", subcore_axis_name="s")
    return pl.kernel(
        _layer_body,
        out_type=jax.ShapeDtypeStruct((NPAD, D), jnp.float32),
        mesh=mesh,
        compiler_params=pltpu.CompilerParams(use_tc_tiling_on_sc=False),
        scratch_types=[
            pltpu.VMEM((CHUNK,), jnp.int32),
            pltpu.VMEM((CHUNK,), jnp.int32),
            pltpu.VMEM((CHUNK,), jnp.float32),
            pltpu.VMEM((CHUNK, D), jnp.float32),
            pltpu.VMEM_SHARED((HPAD, D), jnp.float32),
            pltpu.SemaphoreType.DMA,
        ],
    )(table, src, dst, wgt, zrows)


def _mean4_body(a, b, c, d, o):
    o[...] = (a[...] + b[...] + c[...] + d[...]) * 0.25


def _mean4(t0, t1, t2, t3):
    flat = NPAD * D // 128
    blk = flat // 16
    args = [t.reshape(flat, 128) for t in (t0, t1, t2, t3)]
    spec = pl.BlockSpec((blk, 128), lambda i: (i, 0))
    out = pl.pallas_call(
        _mean4_body,
        grid=(16,),
        in_specs=[spec] * 4,
        out_specs=spec,
        out_shape=jax.ShapeDtypeStruct((flat, 128), jnp.float32),
    )(*args)
    return out.reshape(NPAD, D)


@functools.partial(jax.jit)
def kernel(user_emb, item_emb, edge_index, edge_weight):
    zpad = jnp.zeros((SHIFT, D), jnp.float32)
    table = jnp.concatenate(
        [user_emb, item_emb[: HALF - NUM_USERS], zpad,
         item_emb[HALF - NUM_USERS:], zpad], axis=0)

    n_edges = edge_index.shape[1]
    ep_tile = -(-n_edges // (16 * CHUNK)) * CHUNK
    epad = 16 * ep_tile
    pad = epad - n_edges
    src = jnp.concatenate([edge_index[0], jnp.zeros((pad,), jnp.int32)])
    dst = jnp.concatenate([edge_index[1], jnp.zeros((pad,), jnp.int32)])
    wgt = jnp.concatenate([edge_weight, jnp.zeros((pad,), jnp.float32)])
    zrows = jnp.zeros((ROWS_PER_TILE, D), jnp.float32)

    embs = [table]
    for _ in range(N_LAYERS):
        embs.append(_propagate_layer(embs[-1], src, dst, wgt, zrows))

    light_out = _mean4(*embs)
    users = light_out[:NUM_USERS]
    items = jnp.concatenate(
        [light_out[NUM_USERS:HALF], light_out[HPAD:HPAD + N_NODES - HALF]], axis=0)
    return users, items


# speedup vs baseline: 2.0067x; 1.0008x over previous
import functools

import jax
import jax.numpy as jnp
from jax import lax
from jax.experimental import pallas as pl
from jax.experimental.pallas import tpu as pltpu
from jax.experimental.pallas import tpu_sc as plsc

NUM_USERS = 20000
NUM_ITEMS = 30000
N_NODES = NUM_USERS + NUM_ITEMS
D = 64
N_LAYERS = 3

HALF = N_NODES // 2
HPAD = 25088
NPAD = 2 * HPAD
SHIFT = HPAD - HALF
GARBAGE = HALF + 8

CHUNK = 128
ROWS_PER_TILE = HPAD // 16


def _layer_body(table, src, dst, wgt, zrows, out, idxs, idxd, wv, rows, acc, sem):
    cid = lax.axis_index("c")
    sid = lax.axis_index("s")

    if True:
        base = cid * HALF
        pltpu.sync_copy(zrows, acc.at[pl.ds(sid * ROWS_PER_TILE, ROWS_PER_TILE)])
        plsc.subcore_barrier()

        ep_tile = src.shape[0] // 16
        n_chunks = ep_tile // CHUNK

        def chunk_body(ci, carry):
            off = sid * ep_tile + ci * CHUNK
            pltpu.sync_copy(src.at[pl.ds(off, CHUNK)], idxs)
            pltpu.sync_copy(dst.at[pl.ds(off, CHUNK)], idxd)
            pltpu.sync_copy(wgt.at[pl.ds(off, CHUNK)], wv)
            for j in range(CHUNK // 16):
                s = idxs[pl.ds(j * 16, 16)]
                idxs[pl.ds(j * 16, 16)] = jnp.where(s >= HALF, s + SHIFT, s)
            gcopy = pltpu.async_copy(table.at[idxs], rows, sem)
            lanes = lax.iota(jnp.int32, 16)
            for j in range(CHUNK // 16):
                d = idxd[pl.ds(j * 16, 16)] - base
                ok = (d >= 0) & (d < HALF)
                garbage = HALF + ((lanes + 16 * j) & 63)
                idxd[pl.ds(j * 16, 16)] = jnp.where(ok, d, garbage)
            gcopy.wait()

            def scale_body(j, c2):
                wv16 = wv[pl.ds(j * 16, 16)]
                r0 = j * 16
                for e in range(16):
                    we = wv16[e]
                    for k in range(D // 16):
                        rows[r0 + e, pl.ds(k * 16, 16)] = (
                            rows[r0 + e, pl.ds(k * 16, 16)] * we)
                return c2

            lax.fori_loop(0, CHUNK // 16, scale_body, 0)
            pltpu.sync_copy(rows, acc.at[idxd], add=True)
            return carry

        lax.fori_loop(0, n_chunks, chunk_body, 0)
        plsc.subcore_barrier()
        r0 = sid * ROWS_PER_TILE
        pltpu.sync_copy(acc.at[pl.ds(r0, ROWS_PER_TILE)],
                        out.at[pl.ds(cid * HPAD + r0, ROWS_PER_TILE)])


def _propagate_layer(table, src, dst, wgt, zrows):
    mesh = plsc.VectorSubcoreMesh(core_axis_name="c", subcore_axis_name="s")
    return pl.kernel(
        _layer_body,
        out_type=jax.ShapeDtypeStruct((NPAD, D), jnp.float32),
        mesh=mesh,
        compiler_params=pltpu.CompilerParams(use_tc_tiling_on_sc=False),
        scratch_types=[
            pltpu.VMEM((CHUNK,), jnp.int32),
            pltpu.VMEM((CHUNK,), jnp.int32),
            pltpu.VMEM((CHUNK,), jnp.float32),
            pltpu.VMEM((CHUNK, D), jnp.float32),
            pltpu.VMEM_SHARED((HPAD, D), jnp.float32),
            pltpu.SemaphoreType.DMA,
        ],
    )(table, src, dst, wgt, zrows)


def _mean4_body(a, b, c, d, o):
    o[...] = (a[...] + b[...] + c[...] + d[...]) * 0.25


def _mean4(t0, t1, t2, t3):
    flat = NPAD * D // 128
    blk = flat // 16
    args = [t.reshape(flat, 128) for t in (t0, t1, t2, t3)]
    spec = pl.BlockSpec((blk, 128), lambda i: (i, 0))
    out = pl.pallas_call(
        _mean4_body,
        grid=(16,),
        in_specs=[spec] * 4,
        out_specs=spec,
        out_shape=jax.ShapeDtypeStruct((flat, 128), jnp.float32),
    )(*args)
    return out.reshape(NPAD, D)


@functools.partial(jax.jit)
def kernel(user_emb, item_emb, edge_index, edge_weight):
    zpad = jnp.zeros((SHIFT, D), jnp.float32)
    table = jnp.concatenate(
        [user_emb, item_emb[: HALF - NUM_USERS], zpad,
         item_emb[HALF - NUM_USERS:], zpad], axis=0)

    n_edges = edge_index.shape[1]
    ep_tile = -(-n_edges // (16 * CHUNK)) * CHUNK
    epad = 16 * ep_tile
    pad = epad - n_edges
    src = jnp.concatenate([edge_index[0], jnp.zeros((pad,), jnp.int32)])
    dst = jnp.concatenate([edge_index[1], jnp.zeros((pad,), jnp.int32)])
    wgt = jnp.concatenate([edge_weight, jnp.zeros((pad,), jnp.float32)])
    zrows = jnp.zeros((ROWS_PER_TILE, D), jnp.float32)

    embs = [table]
    for _ in range(N_LAYERS):
        embs.append(_propagate_layer(embs[-1], src, dst, wgt, zrows))

    light_out = _mean4(*embs)
    users = light_out[:NUM_USERS]
    items = jnp.concatenate(
        [light_out[NUM_USERS:HALF], light_out[HPAD:HPAD + N_NODES - HALF]], axis=0)
    return users, items


# speedup vs baseline: 9.9109x; 4.9388x over previous
import functools

import jax
import jax.numpy as jnp
from jax import lax
from jax.experimental import pallas as pl
from jax.experimental.pallas import tpu as pltpu
from jax.experimental.pallas import tpu_sc as plsc

NUM_USERS = 20000
NUM_ITEMS = 30000
N_NODES = NUM_USERS + NUM_ITEMS
D = 64
DH = D // 2
N_LAYERS = 3

CHUNK = 128
BLK = 2
EPB = BLK * CHUNK
ROWS_PER_TILE = N_NODES // 16


def _layer_body(tables, esrc, edst, ewgt, zrows, out, sb, db, wb, rows, acc,
                semi, sem):
    cid = lax.axis_index("c")
    sid = lax.axis_index("s")

    pltpu.sync_copy(zrows, acc.at[pl.ds(sid * ROWS_PER_TILE, ROWS_PER_TILE)])
    plsc.subcore_barrier()

    tab = tables.at[cid]
    ep_tile = esrc.shape[0] // 16
    n_blocks = ep_tile // EPB
    e0 = sid * ep_tile

    def load_block(b, q):
        for j in range(BLK):
            off = e0 + b * EPB + j * CHUNK
            pltpu.async_copy(esrc.at[pl.ds(off, CHUNK)], sb.at[q, j], semi)
            pltpu.async_copy(edst.at[pl.ds(off, CHUNK)], db.at[q, j], semi)
            pltpu.async_copy(ewgt.at[pl.ds(off, CHUNK)], wb.at[q, j], semi)
        for j in range(BLK):
            off = e0 + b * EPB + j * CHUNK
            pltpu.make_async_copy(esrc.at[pl.ds(off, CHUNK)], sb.at[q, j], semi).wait()
            pltpu.make_async_copy(edst.at[pl.ds(off, CHUNK)], db.at[q, j], semi).wait()
            pltpu.make_async_copy(ewgt.at[pl.ds(off, CHUNK)], wb.at[q, j], semi).wait()

    def issue_gather(q, p):
        for j in range(BLK):
            pltpu.async_copy(tab.at[sb.at[q, j]],
                             rows.at[p, pl.ds(j * CHUNK, CHUNK)], sem.at[p])

    def wait_gather(q, p):
        for j in range(BLK):
            pltpu.make_async_copy(tab.at[sb.at[q, j]],
                                  rows.at[p, pl.ds(j * CHUNK, CHUNK)],
                                  sem.at[p]).wait()

    def scale(p, q):
        def g_body(g, c2):
            j = g >> 3
            i = g & 7
            w16 = wb[q, j, pl.ds(i * 16, 16)]
            r0 = g * 16
            for e in range(16):
                we = w16[e]
                for k in range(DH // 16):
                    rows[p, r0 + e, pl.ds(k * 16, 16)] = (
                        rows[p, r0 + e, pl.ds(k * 16, 16)] * we)
            return c2

        lax.fori_loop(0, EPB // 16, g_body, 0)

    def scatter(p, q):
        for j in range(BLK):
            pltpu.sync_copy(rows.at[p, pl.ds(j * CHUNK, CHUNK)],
                            acc.at[db.at[q, j]], add=True)

    load_block(0, 0)
    issue_gather(0, 0)

    def loop_body(b, carry):
        q = b & 1
        qn = 1 - q

        @pl.when(b + 1 < n_blocks)
        def _():
            load_block(b + 1, qn)
            issue_gather(qn, qn)

        wait_gather(q, q)
        scale(q, q)
        scatter(q, q)
        return carry

    lax.fori_loop(0, n_blocks, loop_body, 0)
    plsc.subcore_barrier()
    r0 = sid * ROWS_PER_TILE
    pltpu.sync_copy(acc.at[pl.ds(r0, ROWS_PER_TILE)],
                    out.at[cid, pl.ds(r0, ROWS_PER_TILE)])


def _propagate_layer(tables, esrc, edst, ewgt, zrows):
    mesh = plsc.VectorSubcoreMesh(core_axis_name="c", subcore_axis_name="s")
    return pl.kernel(
        _layer_body,
        out_type=jax.ShapeDtypeStruct((2, N_NODES, DH), jnp.float32),
        mesh=mesh,
        compiler_params=pltpu.CompilerParams(use_tc_tiling_on_sc=False),
        scratch_types=[
            pltpu.VMEM((2, BLK, CHUNK), jnp.int32),
            pltpu.VMEM((2, BLK, CHUNK), jnp.int32),
            pltpu.VMEM((2, BLK, CHUNK), jnp.float32),
            pltpu.VMEM((2, EPB, DH), jnp.float32),
            pltpu.VMEM_SHARED((N_NODES, DH), jnp.float32),
            pltpu.SemaphoreType.DMA,
            pltpu.SemaphoreType.DMA((2,)),
        ],
    )(tables, esrc, edst, ewgt, zrows)


def _mean4_body(a, b, c, d, o):
    o[...] = (a[...] + b[...] + c[...] + d[...]) * 0.25


def _mean4(t0, t1, t2, t3):
    flat = 2 * N_NODES * DH // 128
    blk = flat // 25
    args = [t.reshape(flat, 128) for t in (t0, t1, t2, t3)]
    spec = pl.BlockSpec((blk, 128), lambda i: (i, 0))
    out = pl.pallas_call(
        _mean4_body,
        grid=(25,),
        in_specs=[spec] * 4,
        out_specs=spec,
        out_shape=jax.ShapeDtypeStruct((flat, 128), jnp.float32),
    )(*args)
    return out.reshape(2, N_NODES, DH)


@functools.partial(jax.jit)
def kernel(user_emb, item_emb, edge_index, edge_weight):
    full = jnp.concatenate([user_emb, item_emb], axis=0)
    tables = jnp.stack([full[:, :DH], full[:, DH:]], axis=0)

    n_edges = edge_index.shape[1]
    ep_tile = -(-n_edges // (16 * EPB)) * EPB
    epad = 16 * ep_tile
    pad = epad - n_edges
    src = jnp.concatenate([edge_index[0], jnp.zeros((pad,), jnp.int32)])
    dst = jnp.concatenate([edge_index[1], jnp.zeros((pad,), jnp.int32)])
    wgt = jnp.concatenate([edge_weight, jnp.zeros((pad,), jnp.float32)])

    zrows = jnp.zeros((ROWS_PER_TILE, DH), jnp.float32)

    embs = [tables]
    for _ in range(N_LAYERS):
        embs.append(_propagate_layer(embs[-1], src, dst, wgt, zrows))

    light_out = _mean4(*embs)
    full_out = jnp.concatenate([light_out[0], light_out[1]], axis=1)
    return full_out[:NUM_USERS], full_out[NUM_USERS:]
